# Initial kernel scaffold; baseline (speedup 1.0000x reference)
import jax
import jax.numpy as jnp
from jax.experimental import pallas as pl


def kernel(token_ids, embed_matrix):
    raise NotImplementedError("write your pallas kernel here")


# speedup vs baseline: 1.0945x; 1.0945x over previous
import functools

import jax
import jax.numpy as jnp
from jax import lax
from jax.experimental import pallas as pl
from jax.experimental.pallas import tpu as pltpu
from jax.experimental.pallas import tpu_sc as plsc

_NC = 2
_NS = 16
_NW = _NC * _NS

_IDX_PER_GRP = 128
_GRPS_PER_CHUNK = 8


@functools.cache
def _build(B, V, D):
    groups = B // _IDX_PER_GRP
    grps_per_w = groups // _NW
    chunks = grps_per_w // _GRPS_PER_CHUNK
    rows_per_chunk = _GRPS_PER_CHUNK * _IDX_PER_GRP

    mesh = plsc.VectorSubcoreMesh(
        core_axis_name="c", subcore_axis_name="s",
        num_cores=_NC, num_subcores=_NS)

    @functools.partial(
        pl.kernel,
        out_type=jax.ShapeDtypeStruct((B, D), jnp.float32),
        mesh=mesh,
        scratch_types=[
            pltpu.VMEM((_GRPS_PER_CHUNK, _IDX_PER_GRP), jnp.int32),
            pltpu.VMEM((rows_per_chunk, D), jnp.float32),
            pltpu.SemaphoreType.DMA,
        ],
        compiler_params=pltpu.CompilerParams(use_tc_tiling_on_sc=False),
    )
    def k(idx_hbm, table_hbm, out_hbm, idx_v, rows_v, sem):
        wid = lax.axis_index("s") * _NC + lax.axis_index("c")
        grp0 = wid * grps_per_w

        def chunk_body(i, carry):
            g = grp0 + i * _GRPS_PER_CHUNK
            pltpu.sync_copy(idx_hbm.at[pl.ds(g, _GRPS_PER_CHUNK)], idx_v)
            cps = [
                pltpu.async_copy(
                    table_hbm.at[idx_v.at[j]],
                    rows_v.at[pl.ds(j * _IDX_PER_GRP, _IDX_PER_GRP)],
                    sem)
                for j in range(_GRPS_PER_CHUNK)
            ]
            for cp in cps:
                cp.wait()
            pltpu.sync_copy(
                rows_v, out_hbm.at[pl.ds(g * _IDX_PER_GRP, rows_per_chunk)])
            return carry

        lax.fori_loop(0, chunks, chunk_body, 0)

    return k


def kernel(token_ids, embed_matrix):
    S, T = token_ids.shape
    V, D = embed_matrix.shape
    B = S * T
    idx2d = token_ids.reshape(B // _IDX_PER_GRP, _IDX_PER_GRP).astype(jnp.int32)
    out = _build(B, V, D)(idx2d, embed_matrix)
    return out.reshape(S, T, D)


# speedup vs baseline: 1.1098x; 1.0140x over previous
import functools

import jax
import jax.numpy as jnp
from jax import lax
from jax.experimental import pallas as pl
from jax.experimental.pallas import tpu as pltpu
from jax.experimental.pallas import tpu_sc as plsc

_NC = 2
_NS = 16
_NW = _NC * _NS

_IDX_PER_GRP = 128
_GRPS_PER_CHUNK = 8


@functools.cache
def _build(B, V, D):
    G = _GRPS_PER_CHUNK
    C = G * _IDX_PER_GRP
    grps_per_w = B // _IDX_PER_GRP // _NW
    chunks = grps_per_w // G
    assert chunks * G == grps_per_w and chunks >= 4

    mesh = plsc.VectorSubcoreMesh(
        core_axis_name="c", subcore_axis_name="s",
        num_cores=_NC, num_subcores=_NS)

    @functools.partial(
        pl.kernel,
        out_type=jax.ShapeDtypeStruct((B, D), jnp.float32),
        mesh=mesh,
        scratch_types=[
            pltpu.VMEM((2, G, _IDX_PER_GRP), jnp.int32),
            pltpu.VMEM((2, C, D), jnp.float32),
            pltpu.SemaphoreType.DMA,
            pltpu.SemaphoreType.DMA,
            pltpu.SemaphoreType.DMA,
            pltpu.SemaphoreType.DMA,
            pltpu.SemaphoreType.DMA,
        ],
        compiler_params=pltpu.CompilerParams(use_tc_tiling_on_sc=False),
    )
    def k(idx_hbm, table_hbm, out_hbm, idx_v, rows_v,
          isem0, isem1, osem0, osem1, gsem):
        wid = lax.axis_index("s") * _NC + lax.axis_index("c")
        grp0 = wid * grps_per_w
        isems = (isem0, isem1)
        osems = (osem0, osem1)

        def idx_copy(i, b):
            return pltpu.make_async_copy(
                idx_hbm.at[pl.ds(grp0 + i * G, G)], idx_v.at[b], isems[b])

        def out_copy(i, b):
            return pltpu.make_async_copy(
                rows_v.at[b],
                out_hbm.at[pl.ds((grp0 + i * G) * _IDX_PER_GRP, C)],
                osems[b])

        def step(i, b, wait_prev):
            idx_copy(i, b).wait()
            if wait_prev:
                out_copy(i - 2, b).wait()
            cps = [
                pltpu.async_copy(
                    table_hbm.at[idx_v.at[b].at[j]],
                    rows_v.at[b].at[pl.ds(j * _IDX_PER_GRP, _IDX_PER_GRP)],
                    gsem)
                for j in range(G)
            ]
            for cp in cps:
                cp.wait()
            out_copy(i, b).start()

            @pl.when(i + 2 < chunks)
            def _():
                idx_copy(i + 2, b).start()

        idx_copy(0, 0).start()
        idx_copy(1, 1).start()
        step(0, 0, False)
        step(1, 1, False)

        def body(o, carry):
            step(2 * o, 0, True)
            step(2 * o + 1, 1, True)
            return carry

        lax.fori_loop(1, chunks // 2, body, 0)

        if chunks % 2:
            step(chunks - 1, (chunks - 1) % 2, True)
        out_copy(chunks - 2, (chunks - 2) % 2).wait()
        out_copy(chunks - 1, (chunks - 1) % 2).wait()

    return k


def kernel(token_ids, embed_matrix):
    S, T = token_ids.shape
    V, D = embed_matrix.shape
    B = S * T
    idx2d = token_ids.reshape(B // _IDX_PER_GRP, _IDX_PER_GRP).astype(jnp.int32)
    out = _build(B, V, D)(idx2d, embed_matrix)
    return out.reshape(S, T, D)


# speedup vs baseline: 1.7949x; 1.6173x over previous
import functools

import jax
import jax.numpy as jnp
from jax import lax
from jax.experimental import pallas as pl
from jax.experimental.pallas import tpu as pltpu
from jax.experimental.pallas import tpu_sc as plsc

_NC = 2
_NS = 16
_NW = _NC * _NS

_ROWS_PER_CHUNK = 32


@functools.cache
def _build(S, T, V, D):
    R = _ROWS_PER_CHUNK
    rows_per_w = S // _NW
    chunks = rows_per_w // R
    assert chunks * R == rows_per_w and chunks >= 4

    mesh = plsc.VectorSubcoreMesh(
        core_axis_name="c", subcore_axis_name="s",
        num_cores=_NC, num_subcores=_NS)

    @functools.partial(
        pl.kernel,
        out_type=jax.ShapeDtypeStruct((S, T, D), jnp.float32),
        mesh=mesh,
        scratch_types=[
            pltpu.VMEM((2, R, T), jnp.int32),
            pltpu.VMEM((2, R, T, D), jnp.float32),
            pltpu.SemaphoreType.DMA,
            pltpu.SemaphoreType.DMA,
            pltpu.SemaphoreType.DMA,
            pltpu.SemaphoreType.DMA,
            pltpu.SemaphoreType.DMA,
        ],
        compiler_params=pltpu.CompilerParams(use_tc_tiling_on_sc=False),
    )
    def k(idx_hbm, table_hbm, out_hbm, idx_v, rows_v,
          isem0, isem1, osem0, osem1, gsem):
        wid = lax.axis_index("s") * _NC + lax.axis_index("c")
        row0 = wid * rows_per_w
        isems = (isem0, isem1)
        osems = (osem0, osem1)

        def idx_copy(i, b):
            return pltpu.make_async_copy(
                idx_hbm.at[pl.ds(row0 + i * R, R)], idx_v.at[b], isems[b])

        def out_copy(i, b):
            return pltpu.make_async_copy(
                rows_v.at[b], out_hbm.at[pl.ds(row0 + i * R, R)], osems[b])

        def step(i, b, wait_prev):
            idx_copy(i, b).wait()
            if wait_prev:
                out_copy(i - 2, b).wait()
            cps = [
                pltpu.async_copy(
                    table_hbm.at[idx_v.at[b].at[r]],
                    rows_v.at[b].at[r],
                    gsem)
                for r in range(R)
            ]
            for cp in cps:
                cp.wait()
            out_copy(i, b).start()

            @pl.when(i + 2 < chunks)
            def _():
                idx_copy(i + 2, b).start()

        idx_copy(0, 0).start()
        idx_copy(1, 1).start()
        step(0, 0, False)
        step(1, 1, False)

        def body(o, carry):
            step(2 * o, 0, True)
            step(2 * o + 1, 1, True)
            return carry

        lax.fori_loop(1, chunks // 2, body, 0)

        if chunks % 2:
            step(chunks - 1, (chunks - 1) % 2, True)
        out_copy(chunks - 2, (chunks - 2) % 2).wait()
        out_copy(chunks - 1, (chunks - 1) % 2).wait()

    return k


def kernel(token_ids, embed_matrix):
    S, T = token_ids.shape
    V, D = embed_matrix.shape
    return _build(S, T, V, D)(token_ids.astype(jnp.int32), embed_matrix)
